# baseline (speedup 1.0000x reference)
import functools

import jax
import jax.numpy as jnp
from jax import lax
from jax.experimental import pallas as pl
from jax.experimental.pallas import tpu as pltpu
from jax.experimental.pallas import tpu_sc as plsc

B = 4096
L = 200
D = 128
L_HALF = 104
L_PAD = 2 * L_HALF

NC, NS, LANES = 2, 16, 16
NW = NC * NS
BPW = B // NW
NVR = D // LANES

_sc_mesh = plsc.VectorSubcoreMesh(core_axis_name="c", subcore_axis_name="s")


NBUF = 4
NUNITS = 2 * BPW


@functools.partial(
    pl.kernel,
    out_type=jax.ShapeDtypeStruct((B, D), jnp.float32),
    mesh=_sc_mesh,
    scratch_types=[
        pltpu.VMEM((BPW, 2, L_HALF), jnp.int32),
        pltpu.VMEM((NBUF, L_HALF, D), jnp.float32),
        pltpu.VMEM((BPW, D), jnp.float32),
        [pltpu.SemaphoreType.DMA] * NBUF,
    ],
)
def _sc_pool(ids_hbm, table_hbm, out_hbm, ids_v, ring_v, out_v, sems):
    wid = lax.axis_index("s") * NC + lax.axis_index("c")
    base = wid * BPW
    pltpu.sync_copy(ids_hbm.at[pl.ds(base, BPW)], ids_v)

    zeros = jnp.zeros((LANES,), jnp.float32)

    def zero_body(b, _):
        for j in range(NVR):
            out_v[b, pl.ds(j * LANES, LANES)] = zeros
        return _
    lax.fori_loop(0, BPW, zero_body, None)

    def issue(u, slot):
        b = lax.div(u, 2)
        h = lax.rem(u, 2)
        pltpu.async_copy(table_hbm.at[ids_v.at[b, h]], ring_v.at[slot], sems[slot])

    def reduce_unit(u, slot):
        b = lax.div(u, 2)
        buf = ring_v.at[slot]

        def body(l, accs):
            return tuple(a + buf[l, pl.ds(j * LANES, LANES)]
                         for j, a in enumerate(accs))
        accs = lax.fori_loop(0, L_HALF, body, tuple(zeros for _ in range(NVR)))
        for j, a in enumerate(accs):
            sl = pl.ds(j * LANES, LANES)
            out_v[b, sl] = out_v[b, sl] + a

    for s in range(NBUF):
        issue(s, s)

    def round_body(r, _):
        for s in range(NBUF):
            u = r * NBUF + s
            pltpu.make_async_copy(table_hbm.at[ids_v.at[0, 0]], ring_v.at[s],
                                  sems[s]).wait()
            reduce_unit(u, s)

            @pl.when(u + NBUF < NUNITS)
            def _():
                issue(u + NBUF, s)
        return _

    lax.fori_loop(0, NUNITS // NBUF, round_body, None)
    pltpu.sync_copy(out_v, out_hbm.at[pl.ds(base, BPW)])


_TC_CHUNK = 512


def _mlp_body(ids_ref, x_ref, w0_ref, b0_ref, w1_ref, b1_ref, w2_ref, b2_ref,
              wo_ref, bo_ref, o_ref):
    cnt = jnp.sum((ids_ref[...] != 0).astype(jnp.float32), axis=1, keepdims=True)
    x = x_ref[...] * lax.rsqrt(cnt)
    h = jnp.maximum(jnp.dot(x, w0_ref[...], preferred_element_type=jnp.float32)
                    + b0_ref[...], 0.0)
    h = jnp.maximum(jnp.dot(h, w1_ref[...], preferred_element_type=jnp.float32)
                    + b1_ref[...], 0.0)
    h = jnp.maximum(jnp.dot(h, w2_ref[...], preferred_element_type=jnp.float32)
                    + b2_ref[...], 0.0)
    o_ref[...] = (jnp.dot(h, wo_ref[...], preferred_element_type=jnp.float32)
                  + bo_ref[...])


def _resident(shape):
    return pl.BlockSpec(shape, lambda i: (0,) * len(shape))


_mlp_call = pl.pallas_call(
    _mlp_body,
    grid=(B // _TC_CHUNK,),
    in_specs=[
        pl.BlockSpec((_TC_CHUNK, L), lambda i: (i, 0)),
        pl.BlockSpec((_TC_CHUNK, D), lambda i: (i, 0)),
        _resident((D, 2048)),
        _resident((1, 2048)),
        _resident((2048, 1024)),
        _resident((1, 1024)),
        _resident((1024, 2048)),
        _resident((1, 2048)),
        _resident((2048, D)),
        _resident((1, D)),
    ],
    out_specs=pl.BlockSpec((_TC_CHUNK, D), lambda i: (i, 0)),
    out_shape=jax.ShapeDtypeStruct((B, D), jnp.float32),
)


def kernel(in_item_id, table, W0, b0, W1, b1, W2, b2, W_out, b_out):
    ids = in_item_id.astype(jnp.int32)
    ids_pad = jnp.pad(ids, ((0, 0), (0, L_PAD - L))).reshape(B, 2, L_HALF)
    pooled = _sc_pool(ids_pad, table)
    return _mlp_call(ids, pooled, W0, b0.reshape(1, -1), W1, b1.reshape(1, -1),
                     W2, b2.reshape(1, -1), W_out, b_out.reshape(1, -1))


# speedup vs baseline: 1.0081x; 1.0081x over previous
import functools

import jax
import jax.numpy as jnp
from jax import lax
from jax.experimental import pallas as pl
from jax.experimental.pallas import tpu as pltpu
from jax.experimental.pallas import tpu_sc as plsc

B = 4096
L = 200
D = 128
L_HALF = 104
L_PAD = 2 * L_HALF

NC, NS, LANES = 2, 16, 16
NW = NC * NS
BPW = B // NW
NVR = D // LANES

_sc_mesh = plsc.VectorSubcoreMesh(core_axis_name="c", subcore_axis_name="s")


NCHUNK = 2
BCH = B // NCHUNK
BPWC = BCH // NW
NBUF = 4
NUNITS = 2 * BPWC


@functools.partial(
    pl.kernel,
    out_type=jax.ShapeDtypeStruct((BCH, D), jnp.float32),
    mesh=_sc_mesh,
    scratch_types=[
        pltpu.VMEM((BPWC, 2, L_HALF), jnp.int32),
        pltpu.VMEM((NBUF, L_HALF, D), jnp.float32),
        pltpu.VMEM((BPWC, D), jnp.float32),
        [pltpu.SemaphoreType.DMA] * NBUF,
    ],
)
def _sc_pool(ids_hbm, table_hbm, out_hbm, ids_v, ring_v, out_v, sems):
    wid = lax.axis_index("s") * NC + lax.axis_index("c")
    base = wid * BPWC
    pltpu.sync_copy(ids_hbm.at[pl.ds(base, BPWC)], ids_v)

    zeros = jnp.zeros((LANES,), jnp.float32)

    def zero_body(b, _):
        for j in range(NVR):
            out_v[b, pl.ds(j * LANES, LANES)] = zeros
        return _
    lax.fori_loop(0, BPWC, zero_body, None)

    def issue(u, slot):
        b = lax.div(u, 2)
        h = lax.rem(u, 2)
        pltpu.async_copy(table_hbm.at[ids_v.at[b, h]], ring_v.at[slot], sems[slot])

    def reduce_unit(u, slot):
        b = lax.div(u, 2)
        buf = ring_v.at[slot]

        def body(l, accs):
            return tuple(a + buf[l, pl.ds(j * LANES, LANES)]
                         for j, a in enumerate(accs))
        accs = lax.fori_loop(0, L_HALF, body, tuple(zeros for _ in range(NVR)))
        for j, a in enumerate(accs):
            sl = pl.ds(j * LANES, LANES)
            out_v[b, sl] = out_v[b, sl] + a

    for s in range(NBUF):
        issue(s, s)

    def round_body(r, _):
        for s in range(NBUF):
            u = r * NBUF + s
            pltpu.make_async_copy(table_hbm.at[ids_v.at[0, 0]], ring_v.at[s],
                                  sems[s]).wait()
            reduce_unit(u, s)

            @pl.when(u + NBUF < NUNITS)
            def _():
                issue(u + NBUF, s)
        return _

    lax.fori_loop(0, NUNITS // NBUF, round_body, None)
    pltpu.sync_copy(out_v, out_hbm.at[pl.ds(base, BPWC)])


_TC_CHUNK = 512


def _mlp_body(ids_ref, x_ref, w0_ref, b0_ref, w1_ref, b1_ref, w2_ref, b2_ref,
              wo_ref, bo_ref, o_ref):
    cnt = jnp.sum((ids_ref[...] != 0).astype(jnp.float32), axis=1, keepdims=True)
    x = x_ref[...] * lax.rsqrt(cnt)
    h = jnp.maximum(jnp.dot(x, w0_ref[...], preferred_element_type=jnp.float32)
                    + b0_ref[...], 0.0)
    h = jnp.maximum(jnp.dot(h, w1_ref[...], preferred_element_type=jnp.float32)
                    + b1_ref[...], 0.0)
    h = jnp.maximum(jnp.dot(h, w2_ref[...], preferred_element_type=jnp.float32)
                    + b2_ref[...], 0.0)
    o_ref[...] = (jnp.dot(h, wo_ref[...], preferred_element_type=jnp.float32)
                  + bo_ref[...])


def _resident(shape):
    return pl.BlockSpec(shape, lambda i: (0,) * len(shape))


_mlp_call = pl.pallas_call(
    _mlp_body,
    grid=(BCH // _TC_CHUNK,),
    in_specs=[
        pl.BlockSpec((_TC_CHUNK, L), lambda i: (i, 0)),
        pl.BlockSpec((_TC_CHUNK, D), lambda i: (i, 0)),
        _resident((D, 2048)),
        _resident((1, 2048)),
        _resident((2048, 1024)),
        _resident((1, 1024)),
        _resident((1024, 2048)),
        _resident((1, 2048)),
        _resident((2048, D)),
        _resident((1, D)),
    ],
    out_specs=pl.BlockSpec((_TC_CHUNK, D), lambda i: (i, 0)),
    out_shape=jax.ShapeDtypeStruct((BCH, D), jnp.float32),
)


def kernel(in_item_id, table, W0, b0, W1, b1, W2, b2, W_out, b_out):
    ids = in_item_id.astype(jnp.int32)
    ids_pad = jnp.pad(ids, ((0, 0), (0, L_PAD - L))).reshape(B, 2, L_HALF)
    ws = (W0, b0.reshape(1, -1), W1, b1.reshape(1, -1), W2, b2.reshape(1, -1),
          W_out, b_out.reshape(1, -1))
    pooled, outs = [], []
    for k in range(NCHUNK):
        ids_k = ids_pad[k * BCH:(k + 1) * BCH]
        if pooled:
            ids_k, _ = lax.optimization_barrier((ids_k, pooled[-1]))
        pooled.append(_sc_pool(ids_k, table))
    for k in range(NCHUNK):
        outs.append(_mlp_call(ids[k * BCH:(k + 1) * BCH], pooled[k], *ws))
    return jnp.concatenate(outs)
